# baseline (speedup 1.0000x reference)
import jax
import jax.numpy as jnp
from jax.experimental import pallas as pl
from jax.experimental.pallas import tpu as pltpu
from jax.experimental.pallas import tpu_sc as plsc

D_MODEL = 64
SCALE = 8.0
LANES = 16
WINDOW = 128


def kernel(x, table):
    b, l = x.shape
    n = b * l
    idx = x.reshape(1, n)

    mesh = plsc.VectorSubcoreMesh(core_axis_name="core",
                                  subcore_axis_name="subcore")

    @pl.kernel(out_type=jax.ShapeDtypeStruct((n, D_MODEL), table.dtype),
               mesh=mesh,
               compiler_params=pltpu.CompilerParams(use_tc_tiling_on_sc=False))
    def emb_kernel(table_hbm, idx_hbm, out_hbm):
        def body(i_vmem, o_vmem):
            pltpu.sync_copy(table_hbm.at[i_vmem.at[0]], o_vmem)

            @pl.loop(0, WINDOW)
            def _(r):
                for c in range(0, D_MODEL, LANES):
                    slc = (pl.ds(r, 1), pl.ds(c, LANES))
                    o_vmem.at[*slc][...] = o_vmem.at[*slc][...] * SCALE

        pltpu.emit_pipeline(
            body,
            grid=(n // WINDOW,),
            in_specs=[pl.BlockSpec((1, WINDOW), index_map=lambda i: (0, i))],
            out_specs=[pl.BlockSpec((WINDOW, D_MODEL),
                                    index_map=lambda i: (i, 0))],
            core_axis_name=("core", "subcore"),
            dimension_semantics=(pltpu.PARALLEL,),
        )(idx_hbm, out_hbm)

    out = emb_kernel(table, idx)
    return out.reshape(b, l, D_MODEL)


# speedup vs baseline: 1.0394x; 1.0394x over previous
import jax
import jax.numpy as jnp
from jax.experimental import pallas as pl
from jax.experimental.pallas import tpu as pltpu
from jax.experimental.pallas import tpu_sc as plsc

D_MODEL = 64
SCALE = 8.0
LANES = 16
WINDOW = 512


def kernel(x, table):
    b, l = x.shape
    n = b * l
    idx = x.reshape(1, n)

    mesh = plsc.VectorSubcoreMesh(core_axis_name="core",
                                  subcore_axis_name="subcore")

    @pl.kernel(out_type=jax.ShapeDtypeStruct((n, D_MODEL), table.dtype),
               mesh=mesh,
               compiler_params=pltpu.CompilerParams(use_tc_tiling_on_sc=False))
    def emb_kernel(table_hbm, idx_hbm, out_hbm):
        def body(i_vmem, o_vmem):
            pltpu.sync_copy(table_hbm.at[i_vmem.at[0]], o_vmem)

            @pl.loop(0, WINDOW)
            def _(r):
                for c in range(0, D_MODEL, LANES):
                    slc = (pl.ds(r, 1), pl.ds(c, LANES))
                    o_vmem.at[*slc][...] = o_vmem.at[*slc][...] * SCALE

        pltpu.emit_pipeline(
            body,
            grid=(n // WINDOW,),
            in_specs=[pl.BlockSpec((1, WINDOW), index_map=lambda i: (0, i))],
            out_specs=[pl.BlockSpec((WINDOW, D_MODEL),
                                    index_map=lambda i: (i, 0))],
            core_axis_name=("core", "subcore"),
            dimension_semantics=(pltpu.PARALLEL,),
        )(idx_hbm, out_hbm)

    out = emb_kernel(table, idx)
    return out.reshape(b, l, D_MODEL)


# speedup vs baseline: 1.4347x; 1.3803x over previous
import jax
import jax.numpy as jnp
from jax import lax
from jax.experimental import pallas as pl
from jax.experimental.pallas import tpu as pltpu
from jax.experimental.pallas import tpu_sc as plsc

D_MODEL = 64
SCALE = 8.0
LANES = 16
NCORES = 2
NSUB = 16
NW = NCORES * NSUB
W = 512
NBUF = 3


def kernel(x, table):
    b, l = x.shape
    n = b * l
    idx = x.reshape(n)
    per_w = n // NW
    nchunk = per_w // W

    mesh = plsc.VectorSubcoreMesh(core_axis_name="core",
                                  subcore_axis_name="subcore")

    @pl.kernel(out_type=jax.ShapeDtypeStruct((n, D_MODEL), table.dtype),
               mesh=mesh,
               scratch_types=[
                   pltpu.VMEM((per_w,), jnp.int32),
                   pltpu.VMEM((NBUF, W, D_MODEL), jnp.float32),
                   pltpu.SemaphoreType.DMA((NBUF,)),
                   pltpu.SemaphoreType.DMA((NBUF,)),
               ],
               compiler_params=pltpu.CompilerParams(use_tc_tiling_on_sc=False))
    def emb_kernel(table_hbm, idx_hbm, out_hbm, idx_v, rows_v, gsem, osem):
        wid = lax.axis_index("subcore") * NCORES + lax.axis_index("core")
        base = wid * per_w
        pltpu.sync_copy(idx_hbm.at[pl.ds(base, per_w)], idx_v)

        def gather(c):
            return pltpu.async_copy(
                table_hbm.at[idx_v.at[pl.ds(c * W, W)]],
                rows_v.at[c % NBUF], gsem.at[c % NBUF])

        ghandles = [gather(0), gather(1)]
        ohandles = [None] * NBUF
        for c in range(nchunk):
            bb = c % NBUF
            if c + 2 < nchunk:
                nb = (c + 2) % NBUF
                if ohandles[nb] is not None:
                    ohandles[nb].wait()
                ghandles.append(gather(c + 2))
            ghandles[c].wait()

            @pl.loop(0, W)
            def _(r):
                for col in range(0, D_MODEL, LANES):
                    slc = (pl.ds(r, 1), pl.ds(col, LANES))
                    rows_v.at[bb][slc] = rows_v.at[bb][slc] * SCALE

            ohandles[bb] = pltpu.async_copy(
                rows_v.at[bb], out_hbm.at[pl.ds(base + c * W, W)],
                osem.at[bb])
        for h in ohandles:
            if h is not None:
                h.wait()

    out = emb_kernel(table, idx)
    return out.reshape(b, l, D_MODEL)
